# baseline (speedup 1.0000x reference)
import functools

import jax
import jax.numpy as jnp
from jax import lax
from jax.experimental import pallas as pl
from jax.experimental.pallas import tpu as pltpu
from jax.experimental.pallas import tpu_sc as plsc

N_DONORS = 64
N_CLUSTERS = 32
N_GENES = 2000
N_VARIANTS = 5000
N_VXG = 10000
EPS = 1e-8
_HALF_LOG_2PI = 0.9189385332046727

_BLK = 512
_NW = 32
_SC_PAD = 10240
_ROWS_W = _SC_PAD // _NW


def _lgamma_pos(x):
    q = x * x + 3.0 * x
    p = q * (q + 2.0)
    small = x < 4.0
    z = jnp.where(small, x + 4.0, x)
    zi = 1.0 / z
    zi2 = zi * zi
    ser = zi * (0.08333333333333333 + zi2 * (-0.002777777777777778
                                             + zi2 * 0.0007936507936507937))
    st = (z - 0.5) * jnp.log(z) - z + _HALF_LOG_2PI + ser
    return jnp.where(small, st - jnp.log(p), st)


def _stirling(z):
    zi = 1.0 / z
    zi2 = zi * zi
    ser = zi * (0.08333333333333333 + zi2 * (-0.002777777777777778
                                             + zi2 * 0.0007936507936507937))
    return (z - 0.5) * jnp.log(z) - z + ser


def _lgamma_diff(xa, xb):
    qa = xa * xa + 3.0 * xa
    pa = qa * (qa + 2.0)
    small_a = xa < 4.0
    za = jnp.where(small_a, xa + 4.0, xa)
    qb = xb * xb + 3.0 * xb
    pb = qb * (qb + 2.0)
    small_b = xb < 4.0
    zb = jnp.where(small_b, xb + 4.0, xb)
    num = jnp.where(small_a, pa, 1.0)
    den = jnp.where(small_b, pb, 1.0)
    return _stirling(zb) - _stirling(za) + jnp.log(num / den)


def _prep_body(baseline_ref, dispersion_ref, m_ref):
    disp = jnp.minimum(jnp.exp(dispersion_ref[...]), 20.0)
    tc = 1.0 / disp
    m_ref[0:N_CLUSTERS, :] = baseline_ref[...]
    m_ref[N_CLUSTERS:2 * N_CLUSTERS, :] = tc
    m_ref[2 * N_CLUSTERS:3 * N_CLUSTERS, :] = jnp.log(tc + EPS)
    m_ref[3 * N_CLUSTERS:4 * N_CLUSTERS, :] = _lgamma_pos(tc)


def _sc_gather(mt, genot, gene_idx, sel_idx):
    mesh = plsc.VectorSubcoreMesh(core_axis_name="c", subcore_axis_name="s")

    @functools.partial(
        pl.kernel,
        out_type=[jax.ShapeDtypeStruct((_SC_PAD, 4 * N_CLUSTERS), jnp.float32),
                  jax.ShapeDtypeStruct((_SC_PAD, 128), jnp.float32)],
        mesh=mesh,
        scratch_types=[pltpu.VMEM((_ROWS_W,), jnp.int32),
                       pltpu.VMEM((_ROWS_W,), jnp.int32),
                       pltpu.VMEM((_ROWS_W, 4 * N_CLUSTERS), jnp.float32),
                       pltpu.VMEM((_ROWS_W, 128), jnp.float32),
                       pltpu.SemaphoreType.DMA],
    )
    def k(mt_hbm, genot_hbm, gidx_hbm, sidx_hbm, mg_hbm, g_hbm,
          gidx_v, sidx_v, mrows_v, grows_v, sem):
        wid = lax.axis_index("s") * 2 + lax.axis_index("c")
        base = wid * _ROWS_W
        pltpu.sync_copy(gidx_hbm.at[pl.ds(base, _ROWS_W)], gidx_v)
        pltpu.sync_copy(sidx_hbm.at[pl.ds(base, _ROWS_W)], sidx_v)
        for off, sz in ((0, 128), (128, 128), (256, 64)):
            pltpu.async_copy(mt_hbm.at[gidx_v.at[pl.ds(off, sz)]],
                             mrows_v.at[pl.ds(off, sz)], sem).wait()
            pltpu.async_copy(genot_hbm.at[sidx_v.at[pl.ds(off, sz)]],
                             grows_v.at[pl.ds(off, sz)], sem).wait()
        pltpu.sync_copy(mrows_v, mg_hbm.at[pl.ds(base, _ROWS_W)])
        pltpu.sync_copy(grows_v, g_hbm.at[pl.ds(base, _ROWS_W)])

    return k(mt, genot, gene_idx, sel_idx)


def _main_body(lidx_ref, fc_ref, mgt_ref, gt_ref, obs_ref,
               lib_ref, expressed_ref, elbo_ref):
    B = fc_ref.shape[-1]
    lidx = lidx_ref[0]

    iota_gene = lax.broadcasted_iota(jnp.int32, (N_GENES, B), 0)
    oh_lg = (iota_gene == lidx).astype(jnp.bfloat16)
    ident = (lax.broadcasted_iota(jnp.int32, (B, B), 0)
             == lax.broadcasted_iota(jnp.int32, (B, B), 1)).astype(jnp.float32)

    dnt = (((0,), (0,)), ((), ()))
    mg = lax.dot_general(mgt_ref[...], ident, dnt,
                         precision=lax.Precision.HIGHEST,
                         preferred_element_type=jnp.float32)
    g = lax.dot_general(gt_ref[:, 0:N_DONORS], ident, dnt,
                        precision=lax.Precision.HIGHEST,
                        preferred_element_type=jnp.float32)
    dn = (((1,), (0,)), ((), ()))
    value = lax.dot_general(obs_ref[...], oh_lg, dn,
                            preferred_element_type=jnp.float32)
    value = value.reshape(N_DONORS, N_CLUSTERS, B)

    baseline_g = mg[0:N_CLUSTERS]
    tc = mg[N_CLUSTERS:2 * N_CLUSTERS]
    l1 = mg[2 * N_CLUSTERS:3 * N_CLUSTERS]
    g0 = mg[3 * N_CLUSTERS:4 * N_CLUSTERS]

    el = baseline_g[None, :, :] + g[:, None, :] * fc_ref[...][None, :, :]
    expressed = jnp.exp(el) * lib_ref[...][:, :, None]
    expressed_ref[...] = expressed

    logits = jnp.log(expressed + EPS) - l1[None, :, :]
    sp = jnp.maximum(logits, 0.0) + jnp.log(1.0 + jnp.exp(-jnp.abs(logits)))
    tcv = tc[None, :, :] + value
    elbo = (tcv * sp - value * logits
            + _lgamma_diff(tcv, 1.0 + value) + g0[None, :, :])
    elbo_ref[...] = elbo


def kernel(fc_log, genotypes, expression_obs, variantxgene_to_gene,
           local_variant_to_local_variantxgene_selector, variantxgene_to_local_gene,
           lib, baseline_log, dispersion_log):
    nblk = _SC_PAD // _BLK
    pad = _SC_PAD - N_VXG

    m = pl.pallas_call(
        _prep_body,
        out_shape=jax.ShapeDtypeStruct((4 * N_CLUSTERS, N_GENES), jnp.float32),
    )(baseline_log, dispersion_log)

    gene_idx = jnp.pad(variantxgene_to_gene.astype(jnp.int32), (0, pad))
    sel_idx = jnp.pad(local_variant_to_local_variantxgene_selector.astype(jnp.int32), (0, pad))
    genot_pad = jnp.pad(genotypes.T, ((0, 0), (0, 128 - N_DONORS)))
    mg_all, g_all = _sc_gather(m.T, genot_pad, gene_idx, sel_idx)

    lidx = jnp.pad(variantxgene_to_local_gene.astype(jnp.int32), (0, pad)).reshape(nblk, 1, _BLK)
    obs_bf = expression_obs.reshape(N_DONORS * N_CLUSTERS, N_GENES).astype(jnp.bfloat16)

    grid = (nblk,)
    out_specs = [
        pl.BlockSpec((N_DONORS, N_CLUSTERS, _BLK), lambda j: (0, 0, j)),
        pl.BlockSpec((N_DONORS, N_CLUSTERS, _BLK), lambda j: (0, 0, j)),
    ]
    in_specs = [
        pl.BlockSpec((1, 1, _BLK), lambda j: (j, 0, 0)),
        pl.BlockSpec((N_CLUSTERS, _BLK), lambda j: (0, j)),
        pl.BlockSpec((_BLK, 4 * N_CLUSTERS), lambda j: (j, 0)),
        pl.BlockSpec((_BLK, 128), lambda j: (j, 0)),
        pl.BlockSpec((N_DONORS * N_CLUSTERS, N_GENES), lambda j: (0, 0)),
        pl.BlockSpec((N_DONORS, N_CLUSTERS), lambda j: (0, 0)),
    ]
    expressed, elbo = pl.pallas_call(
        _main_body,
        grid=grid,
        in_specs=in_specs,
        out_specs=out_specs,
        out_shape=[
            jax.ShapeDtypeStruct((N_DONORS, N_CLUSTERS, N_VXG), jnp.float32),
            jax.ShapeDtypeStruct((N_DONORS, N_CLUSTERS, N_VXG), jnp.float32),
        ],
    )(lidx, fc_log, mg_all, g_all, obs_bf, lib)
    return expressed, elbo


# speedup vs baseline: 1.0297x; 1.0297x over previous
import functools

import jax
import jax.numpy as jnp
from jax import lax
from jax.experimental import pallas as pl
from jax.experimental.pallas import tpu as pltpu
from jax.experimental.pallas import tpu_sc as plsc

N_DONORS = 64
N_CLUSTERS = 32
N_GENES = 2000
N_VARIANTS = 5000
N_VXG = 10000
EPS = 1e-8
_HALF_LOG_2PI = 0.9189385332046727

_BLK = 512
_NW = 32
_SC_PAD = 10240
_ROWS_W = _SC_PAD // _NW


def _lgamma_pos(x):
    q = x * x + 3.0 * x
    p = q * (q + 2.0)
    small = x < 4.0
    z = jnp.where(small, x + 4.0, x)
    zi = 1.0 / z
    zi2 = zi * zi
    ser = zi * (0.08333333333333333 + zi2 * (-0.002777777777777778
                                             + zi2 * 0.0007936507936507937))
    st = (z - 0.5) * jnp.log(z) - z + _HALF_LOG_2PI + ser
    return jnp.where(small, st - jnp.log(p), st)


def _stirling(z):
    zi = 1.0 / z
    zi2 = zi * zi
    ser = zi * (0.08333333333333333 + zi2 * (-0.002777777777777778
                                             + zi2 * 0.0007936507936507937))
    return (z - 0.5) * jnp.log(z) - z + ser


def _lgamma_diff(xa, xb):
    qa = xa * xa + 3.0 * xa
    pa = qa * (qa + 2.0)
    small_a = xa < 4.0
    za = jnp.where(small_a, xa + 4.0, xa)
    qb = xb * xb + 3.0 * xb
    pb = qb * (qb + 2.0)
    small_b = xb < 4.0
    zb = jnp.where(small_b, xb + 4.0, xb)
    num = jnp.where(small_a, pa, 1.0)
    den = jnp.where(small_b, pb, 1.0)
    return _stirling(zb) - _stirling(za) + jnp.log(num / den)


def _prep_body(baseline_ref, dispersion_ref, m_ref):
    disp = jnp.minimum(jnp.exp(dispersion_ref[...]), 20.0)
    tc = 1.0 / disp
    m_ref[0:N_CLUSTERS, :] = baseline_ref[...]
    m_ref[N_CLUSTERS:2 * N_CLUSTERS, :] = tc
    m_ref[2 * N_CLUSTERS:3 * N_CLUSTERS, :] = jnp.log(tc + EPS)
    m_ref[3 * N_CLUSTERS:4 * N_CLUSTERS, :] = _lgamma_pos(tc)


def _sc_gather(mt, genot, gene_idx, sel_idx):
    mesh = plsc.VectorSubcoreMesh(core_axis_name="c", subcore_axis_name="s")

    @functools.partial(
        pl.kernel,
        out_type=[jax.ShapeDtypeStruct((_SC_PAD, 4 * N_CLUSTERS), jnp.float32),
                  jax.ShapeDtypeStruct((_SC_PAD, 128), jnp.float32)],
        mesh=mesh,
        scratch_types=[pltpu.VMEM((_ROWS_W,), jnp.int32),
                       pltpu.VMEM((_ROWS_W,), jnp.int32),
                       pltpu.VMEM((_ROWS_W, 4 * N_CLUSTERS), jnp.float32),
                       pltpu.VMEM((_ROWS_W, 128), jnp.float32),
                       pltpu.SemaphoreType.DMA],
    )
    def k(mt_hbm, genot_hbm, gidx_hbm, sidx_hbm, mg_hbm, g_hbm,
          gidx_v, sidx_v, mrows_v, grows_v, sem):
        wid = lax.axis_index("s") * 2 + lax.axis_index("c")
        base = wid * _ROWS_W
        pltpu.sync_copy(gidx_hbm.at[pl.ds(base, _ROWS_W)], gidx_v)
        pltpu.sync_copy(sidx_hbm.at[pl.ds(base, _ROWS_W)], sidx_v)
        copies = []
        for off, sz in ((0, 128), (128, 128), (256, 64)):
            copies.append(pltpu.async_copy(mt_hbm.at[gidx_v.at[pl.ds(off, sz)]],
                                           mrows_v.at[pl.ds(off, sz)], sem))
            copies.append(pltpu.async_copy(genot_hbm.at[sidx_v.at[pl.ds(off, sz)]],
                                           grows_v.at[pl.ds(off, sz)], sem))
        for c in copies:
            c.wait()
        pltpu.sync_copy(mrows_v, mg_hbm.at[pl.ds(base, _ROWS_W)])
        pltpu.sync_copy(grows_v, g_hbm.at[pl.ds(base, _ROWS_W)])

    return k(mt, genot, gene_idx, sel_idx)


def _main_body(lidx_ref, fc_ref, mgt_ref, gt_ref, obs_ref,
               lib_ref, expressed_ref, elbo_ref):
    B = fc_ref.shape[-1]
    lidx = lidx_ref[0]

    iota_gene = lax.broadcasted_iota(jnp.int32, (N_GENES, B), 0)
    oh_lg = (iota_gene == lidx).astype(jnp.bfloat16)
    ident = (lax.broadcasted_iota(jnp.int32, (B, B), 0)
             == lax.broadcasted_iota(jnp.int32, (B, B), 1)).astype(jnp.float32)

    dnt = (((0,), (0,)), ((), ()))
    mg = lax.dot_general(mgt_ref[...], ident, dnt,
                         precision=lax.Precision.HIGHEST,
                         preferred_element_type=jnp.float32)
    g = lax.dot_general(gt_ref[:, 0:N_DONORS], ident, dnt,
                        precision=lax.Precision.HIGHEST,
                        preferred_element_type=jnp.float32)
    dn = (((1,), (0,)), ((), ()))
    value = lax.dot_general(obs_ref[...], oh_lg, dn,
                            preferred_element_type=jnp.float32)
    value = value.reshape(N_DONORS, N_CLUSTERS, B)

    baseline_g = mg[0:N_CLUSTERS]
    tc = mg[N_CLUSTERS:2 * N_CLUSTERS]
    l1 = mg[2 * N_CLUSTERS:3 * N_CLUSTERS]
    g0 = mg[3 * N_CLUSTERS:4 * N_CLUSTERS]

    el = baseline_g[None, :, :] + g[:, None, :] * fc_ref[...][None, :, :]
    expressed = jnp.exp(el) * lib_ref[...][:, :, None]
    expressed_ref[...] = expressed

    logits = jnp.log(expressed + EPS) - l1[None, :, :]
    sp = jnp.maximum(logits, 0.0) + jnp.log(1.0 + jnp.exp(-jnp.abs(logits)))
    tcv = tc[None, :, :] + value
    elbo = (tcv * sp - value * logits
            + _lgamma_diff(tcv, 1.0 + value) + g0[None, :, :])
    elbo_ref[...] = elbo


def kernel(fc_log, genotypes, expression_obs, variantxgene_to_gene,
           local_variant_to_local_variantxgene_selector, variantxgene_to_local_gene,
           lib, baseline_log, dispersion_log):
    nblk = _SC_PAD // _BLK
    pad = _SC_PAD - N_VXG

    m = pl.pallas_call(
        _prep_body,
        out_shape=jax.ShapeDtypeStruct((4 * N_CLUSTERS, N_GENES), jnp.float32),
    )(baseline_log, dispersion_log)

    gene_idx = jnp.pad(variantxgene_to_gene.astype(jnp.int32), (0, pad))
    sel_idx = jnp.pad(local_variant_to_local_variantxgene_selector.astype(jnp.int32), (0, pad))
    genot_pad = jnp.pad(genotypes.T, ((0, 0), (0, 128 - N_DONORS)))
    mg_all, g_all = _sc_gather(m.T, genot_pad, gene_idx, sel_idx)

    lidx = jnp.pad(variantxgene_to_local_gene.astype(jnp.int32), (0, pad)).reshape(nblk, 1, _BLK)
    obs_bf = expression_obs.reshape(N_DONORS * N_CLUSTERS, N_GENES).astype(jnp.bfloat16)

    grid = (nblk,)
    out_specs = [
        pl.BlockSpec((N_DONORS, N_CLUSTERS, _BLK), lambda j: (0, 0, j)),
        pl.BlockSpec((N_DONORS, N_CLUSTERS, _BLK), lambda j: (0, 0, j)),
    ]
    in_specs = [
        pl.BlockSpec((1, 1, _BLK), lambda j: (j, 0, 0)),
        pl.BlockSpec((N_CLUSTERS, _BLK), lambda j: (0, j)),
        pl.BlockSpec((_BLK, 4 * N_CLUSTERS), lambda j: (j, 0)),
        pl.BlockSpec((_BLK, 128), lambda j: (j, 0)),
        pl.BlockSpec((N_DONORS * N_CLUSTERS, N_GENES), lambda j: (0, 0)),
        pl.BlockSpec((N_DONORS, N_CLUSTERS), lambda j: (0, 0)),
    ]
    expressed, elbo = pl.pallas_call(
        _main_body,
        grid=grid,
        in_specs=in_specs,
        out_specs=out_specs,
        out_shape=[
            jax.ShapeDtypeStruct((N_DONORS, N_CLUSTERS, N_VXG), jnp.float32),
            jax.ShapeDtypeStruct((N_DONORS, N_CLUSTERS, N_VXG), jnp.float32),
        ],
    )(lidx, fc_log, mg_all, g_all, obs_bf, lib)
    return expressed, elbo


# speedup vs baseline: 1.1030x; 1.0712x over previous
import functools

import jax
import jax.numpy as jnp
from jax import lax
from jax.experimental import pallas as pl
from jax.experimental.pallas import tpu as pltpu
from jax.experimental.pallas import tpu_sc as plsc

N_DONORS = 64
N_CLUSTERS = 32
N_GENES = 2000
N_VARIANTS = 5000
N_VXG = 10000
EPS = 1e-8
_HALF_LOG_2PI = 0.9189385332046727

_BLK = 512
_NW = 32
_SC_PAD = 10240
_ROWS_W = _SC_PAD // _NW


def _lgamma_pos(x):
    q = x * x + 3.0 * x
    p = q * (q + 2.0)
    small = x < 4.0
    z = jnp.where(small, x + 4.0, x)
    zi = 1.0 / z
    zi2 = zi * zi
    ser = zi * (0.08333333333333333 + zi2 * (-0.002777777777777778
                                             + zi2 * 0.0007936507936507937))
    st = (z - 0.5) * jnp.log(z) - z + _HALF_LOG_2PI + ser
    return jnp.where(small, st - jnp.log(p), st)


def _stirling(z):
    zi = 1.0 / z
    zi2 = zi * zi
    ser = zi * (0.08333333333333333 + zi2 * (-0.002777777777777778
                                             + zi2 * 0.0007936507936507937))
    return (z - 0.5) * jnp.log(z) - z + ser


def _lgamma_diff(xa, xb):
    qa = xa * xa + 3.0 * xa
    pa = qa * (qa + 2.0)
    qb = xb * xb + 3.0 * xb
    pb = qb * (qb + 2.0)
    return _stirling(xb + 4.0) - _stirling(xa + 4.0) + jnp.log(pa / pb)


def _prep_body(baseline_ref, dispersion_ref, m_ref):
    disp = jnp.minimum(jnp.exp(dispersion_ref[...]), 20.0)
    tc = 1.0 / disp
    m_ref[0:N_CLUSTERS, :] = baseline_ref[...]
    m_ref[N_CLUSTERS:2 * N_CLUSTERS, :] = tc
    m_ref[2 * N_CLUSTERS:3 * N_CLUSTERS, :] = jnp.log(tc + EPS)
    m_ref[3 * N_CLUSTERS:4 * N_CLUSTERS, :] = _lgamma_pos(tc)


def _sc_gather(mt, genot, gene_idx, sel_idx):
    mesh = plsc.VectorSubcoreMesh(core_axis_name="c", subcore_axis_name="s")

    @functools.partial(
        pl.kernel,
        out_type=[jax.ShapeDtypeStruct((_SC_PAD, 4 * N_CLUSTERS), jnp.float32),
                  jax.ShapeDtypeStruct((_SC_PAD, 128), jnp.float32)],
        mesh=mesh,
        scratch_types=[pltpu.VMEM((_ROWS_W,), jnp.int32),
                       pltpu.VMEM((_ROWS_W,), jnp.int32),
                       pltpu.VMEM((_ROWS_W, 4 * N_CLUSTERS), jnp.float32),
                       pltpu.VMEM((_ROWS_W, 128), jnp.float32),
                       pltpu.SemaphoreType.DMA],
    )
    def k(mt_hbm, genot_hbm, gidx_hbm, sidx_hbm, mg_hbm, g_hbm,
          gidx_v, sidx_v, mrows_v, grows_v, sem):
        wid = lax.axis_index("s") * 2 + lax.axis_index("c")
        base = wid * _ROWS_W
        pltpu.sync_copy(gidx_hbm.at[pl.ds(base, _ROWS_W)], gidx_v)
        pltpu.sync_copy(sidx_hbm.at[pl.ds(base, _ROWS_W)], sidx_v)
        copies = []
        for off, sz in ((0, 128), (128, 128), (256, 64)):
            copies.append(pltpu.async_copy(mt_hbm.at[gidx_v.at[pl.ds(off, sz)]],
                                           mrows_v.at[pl.ds(off, sz)], sem))
            copies.append(pltpu.async_copy(genot_hbm.at[sidx_v.at[pl.ds(off, sz)]],
                                           grows_v.at[pl.ds(off, sz)], sem))
        for c in copies:
            c.wait()
        pltpu.sync_copy(mrows_v, mg_hbm.at[pl.ds(base, _ROWS_W)])
        pltpu.sync_copy(grows_v, g_hbm.at[pl.ds(base, _ROWS_W)])

    return k(mt, genot, gene_idx, sel_idx)


def _main_body(lidx_ref, fc_ref, mgt_ref, gt_ref, obs_ref,
               lib_ref, expressed_ref, elbo_ref):
    B = fc_ref.shape[-1]
    lidx = lidx_ref[0]

    iota_gene = lax.broadcasted_iota(jnp.int32, (N_GENES, B), 0)
    oh_lg = (iota_gene == lidx).astype(jnp.bfloat16)
    ident = (lax.broadcasted_iota(jnp.int32, (B, B), 0)
             == lax.broadcasted_iota(jnp.int32, (B, B), 1)).astype(jnp.float32)

    dnt = (((0,), (0,)), ((), ()))
    mg = lax.dot_general(mgt_ref[...], ident, dnt,
                         precision=lax.Precision.HIGHEST,
                         preferred_element_type=jnp.float32)
    g = lax.dot_general(gt_ref[:, 0:N_DONORS], ident, dnt,
                        precision=lax.Precision.HIGHEST,
                        preferred_element_type=jnp.float32)
    dn = (((1,), (0,)), ((), ()))
    value = lax.dot_general(obs_ref[...], oh_lg, dn,
                            preferred_element_type=jnp.float32)
    value = value.reshape(N_DONORS, N_CLUSTERS, B)

    baseline_g = mg[0:N_CLUSTERS]
    tc = mg[N_CLUSTERS:2 * N_CLUSTERS]
    l1 = mg[2 * N_CLUSTERS:3 * N_CLUSTERS]
    g0 = mg[3 * N_CLUSTERS:4 * N_CLUSTERS]

    el = baseline_g[None, :, :] + g[:, None, :] * fc_ref[...][None, :, :]
    expressed = jnp.exp(el) * lib_ref[...][:, :, None]
    expressed_ref[...] = expressed

    logits = jnp.log(expressed + EPS) - l1[None, :, :]
    sp = jnp.log(1.0 + jnp.exp(logits))
    tcv = tc[None, :, :] + value
    elbo = (tcv * sp - value * logits
            + _lgamma_diff(tcv, 1.0 + value) + g0[None, :, :])
    elbo_ref[...] = elbo


def kernel(fc_log, genotypes, expression_obs, variantxgene_to_gene,
           local_variant_to_local_variantxgene_selector, variantxgene_to_local_gene,
           lib, baseline_log, dispersion_log):
    nblk = _SC_PAD // _BLK
    pad = _SC_PAD - N_VXG

    m = pl.pallas_call(
        _prep_body,
        out_shape=jax.ShapeDtypeStruct((4 * N_CLUSTERS, N_GENES), jnp.float32),
    )(baseline_log, dispersion_log)

    gene_idx = jnp.pad(variantxgene_to_gene.astype(jnp.int32), (0, pad))
    sel_idx = jnp.pad(local_variant_to_local_variantxgene_selector.astype(jnp.int32), (0, pad))
    genot_pad = jnp.pad(genotypes.T, ((0, 0), (0, 128 - N_DONORS)))
    mg_all, g_all = _sc_gather(m.T, genot_pad, gene_idx, sel_idx)

    lidx = jnp.pad(variantxgene_to_local_gene.astype(jnp.int32), (0, pad)).reshape(nblk, 1, _BLK)
    obs_bf = expression_obs.reshape(N_DONORS * N_CLUSTERS, N_GENES).astype(jnp.bfloat16)

    grid = (nblk,)
    out_specs = [
        pl.BlockSpec((N_DONORS, N_CLUSTERS, _BLK), lambda j: (0, 0, j)),
        pl.BlockSpec((N_DONORS, N_CLUSTERS, _BLK), lambda j: (0, 0, j)),
    ]
    in_specs = [
        pl.BlockSpec((1, 1, _BLK), lambda j: (j, 0, 0)),
        pl.BlockSpec((N_CLUSTERS, _BLK), lambda j: (0, j)),
        pl.BlockSpec((_BLK, 4 * N_CLUSTERS), lambda j: (j, 0)),
        pl.BlockSpec((_BLK, 128), lambda j: (j, 0)),
        pl.BlockSpec((N_DONORS * N_CLUSTERS, N_GENES), lambda j: (0, 0)),
        pl.BlockSpec((N_DONORS, N_CLUSTERS), lambda j: (0, 0)),
    ]
    expressed, elbo = pl.pallas_call(
        _main_body,
        grid=grid,
        in_specs=in_specs,
        out_specs=out_specs,
        out_shape=[
            jax.ShapeDtypeStruct((N_DONORS, N_CLUSTERS, N_VXG), jnp.float32),
            jax.ShapeDtypeStruct((N_DONORS, N_CLUSTERS, N_VXG), jnp.float32),
        ],
    )(lidx, fc_log, mg_all, g_all, obs_bf, lib)
    return expressed, elbo


# speedup vs baseline: 1.1271x; 1.0218x over previous
import functools

import jax
import jax.numpy as jnp
from jax import lax
from jax.experimental import pallas as pl
from jax.experimental.pallas import tpu as pltpu
from jax.experimental.pallas import tpu_sc as plsc

N_DONORS = 64
N_CLUSTERS = 32
N_GENES = 2000
N_VARIANTS = 5000
N_VXG = 10000
EPS = 1e-8
_HALF_LOG_2PI = 0.9189385332046727

_BLK = 512
_NW = 32
_SC_PAD = 10240
_HALF = _SC_PAD // 2
_ROWS_W = _HALF // _NW
_CHUNKS = ((0, 128), (128, 32))


def _lgamma_pos(x):
    q = x * x + 3.0 * x
    p = q * (q + 2.0)
    small = x < 4.0
    z = jnp.where(small, x + 4.0, x)
    zi = 1.0 / z
    zi2 = zi * zi
    ser = zi * (0.08333333333333333 + zi2 * (-0.002777777777777778
                                             + zi2 * 0.0007936507936507937))
    st = (z - 0.5) * jnp.log(z) - z + _HALF_LOG_2PI + ser
    return jnp.where(small, st - jnp.log(p), st)


def _stirling(z):
    zi = 1.0 / z
    zi2 = zi * zi
    ser = zi * (0.08333333333333333 + zi2 * (-0.002777777777777778
                                             + zi2 * 0.0007936507936507937))
    return (z - 0.5) * jnp.log(z) - z + ser


def _lgamma_diff(xa, xb):
    qa = xa * xa + 3.0 * xa
    pa = qa * (qa + 2.0)
    qb = xb * xb + 3.0 * xb
    pb = qb * (qb + 2.0)
    return _stirling(xb + 4.0) - _stirling(xa + 4.0) + jnp.log(pa / pb)


def _prep_body(baseline_ref, dispersion_ref, m_ref):
    disp = jnp.minimum(jnp.exp(dispersion_ref[...]), 20.0)
    tc = 1.0 / disp
    m_ref[0:N_CLUSTERS, :] = baseline_ref[...]
    m_ref[N_CLUSTERS:2 * N_CLUSTERS, :] = tc
    m_ref[2 * N_CLUSTERS:3 * N_CLUSTERS, :] = jnp.log(tc + EPS)
    m_ref[3 * N_CLUSTERS:4 * N_CLUSTERS, :] = _lgamma_pos(tc)


def _sc_gather(mt, genot, gene_idx, sel_idx):
    mesh = plsc.VectorSubcoreMesh(core_axis_name="c", subcore_axis_name="s")

    @functools.partial(
        pl.kernel,
        out_type=[jax.ShapeDtypeStruct((_HALF, 4 * N_CLUSTERS), jnp.float32),
                  jax.ShapeDtypeStruct((_HALF, 128), jnp.float32)],
        mesh=mesh,
        scratch_types=[pltpu.VMEM((_ROWS_W,), jnp.int32),
                       pltpu.VMEM((_ROWS_W,), jnp.int32),
                       pltpu.VMEM((_ROWS_W, 4 * N_CLUSTERS), jnp.float32),
                       pltpu.VMEM((_ROWS_W, 128), jnp.float32),
                       pltpu.SemaphoreType.DMA],
    )
    def k(mt_hbm, genot_hbm, gidx_hbm, sidx_hbm, mg_hbm, g_hbm,
          gidx_v, sidx_v, mrows_v, grows_v, sem):
        wid = lax.axis_index("s") * 2 + lax.axis_index("c")
        base = wid * _ROWS_W
        pltpu.sync_copy(gidx_hbm.at[pl.ds(base, _ROWS_W)], gidx_v)
        pltpu.sync_copy(sidx_hbm.at[pl.ds(base, _ROWS_W)], sidx_v)
        copies = []
        for off, sz in _CHUNKS:
            copies.append(pltpu.async_copy(mt_hbm.at[gidx_v.at[pl.ds(off, sz)]],
                                           mrows_v.at[pl.ds(off, sz)], sem))
            copies.append(pltpu.async_copy(genot_hbm.at[sidx_v.at[pl.ds(off, sz)]],
                                           grows_v.at[pl.ds(off, sz)], sem))
        for c in copies:
            c.wait()
        pltpu.sync_copy(mrows_v, mg_hbm.at[pl.ds(base, _ROWS_W)])
        pltpu.sync_copy(grows_v, g_hbm.at[pl.ds(base, _ROWS_W)])

    return k(mt, genot, gene_idx, sel_idx)


def _main_body(lidx_ref, fc_ref, mgt_ref, gt_ref, obs_ref,
               lib_ref, *rest):
    expressed_ref, elbo_ref = rest[-2], rest[-1]
    B = fc_ref.shape[-1]
    lidx = lidx_ref[0]

    iota_gene = lax.broadcasted_iota(jnp.int32, (N_GENES, B), 0)
    oh_lg = (iota_gene == lidx).astype(jnp.bfloat16)
    ident = (lax.broadcasted_iota(jnp.int32, (B, B), 0)
             == lax.broadcasted_iota(jnp.int32, (B, B), 1)).astype(jnp.float32)

    dnt = (((0,), (0,)), ((), ()))
    mg = lax.dot_general(mgt_ref[...], ident, dnt,
                         precision=lax.Precision.HIGHEST,
                         preferred_element_type=jnp.float32)
    g = lax.dot_general(gt_ref[:, 0:N_DONORS], ident, dnt,
                        precision=lax.Precision.HIGHEST,
                        preferred_element_type=jnp.float32)
    dn = (((1,), (0,)), ((), ()))
    value = lax.dot_general(obs_ref[...], oh_lg, dn,
                            preferred_element_type=jnp.float32)
    value = value.reshape(N_DONORS, N_CLUSTERS, B)

    baseline_g = mg[0:N_CLUSTERS]
    tc = mg[N_CLUSTERS:2 * N_CLUSTERS]
    l1 = mg[2 * N_CLUSTERS:3 * N_CLUSTERS]
    g0 = mg[3 * N_CLUSTERS:4 * N_CLUSTERS]

    el = baseline_g[None, :, :] + g[:, None, :] * fc_ref[...][None, :, :]
    expressed = jnp.exp(el) * lib_ref[...][:, :, None]
    expressed_ref[...] = expressed

    logits = jnp.log(expressed + EPS) - l1[None, :, :]
    sp = jnp.log(1.0 + jnp.exp(logits))
    tcv = tc[None, :, :] + value
    elbo = (tcv * sp - value * logits
            + _lgamma_diff(tcv, 1.0 + value) + g0[None, :, :])
    elbo_ref[...] = elbo


def kernel(fc_log, genotypes, expression_obs, variantxgene_to_gene,
           local_variant_to_local_variantxgene_selector, variantxgene_to_local_gene,
           lib, baseline_log, dispersion_log):
    nblk = _SC_PAD // _BLK
    pad = _SC_PAD - N_VXG

    m = pl.pallas_call(
        _prep_body,
        out_shape=jax.ShapeDtypeStruct((4 * N_CLUSTERS, N_GENES), jnp.float32),
    )(baseline_log, dispersion_log)

    gene_idx = jnp.pad(variantxgene_to_gene.astype(jnp.int32), (0, pad))
    sel_idx = jnp.pad(local_variant_to_local_variantxgene_selector.astype(jnp.int32), (0, pad))
    genot_pad = jnp.pad(genotypes.T, ((0, 0), (0, 128 - N_DONORS)))
    mt = m.T
    mg_1, g_1 = _sc_gather(mt, genot_pad, gene_idx[:_HALF], sel_idx[:_HALF])
    mg_2, g_2 = _sc_gather(mt, genot_pad, gene_idx[_HALF:], sel_idx[_HALF:])

    lidx = jnp.pad(variantxgene_to_local_gene.astype(jnp.int32), (0, pad)).reshape(nblk, 1, _BLK)
    obs_bf = expression_obs.reshape(N_DONORS * N_CLUSTERS, N_GENES).astype(jnp.bfloat16)

    hblk = _HALF // _BLK
    out_shape = [
        jax.ShapeDtypeStruct((N_DONORS, N_CLUSTERS, N_VXG), jnp.float32),
        jax.ShapeDtypeStruct((N_DONORS, N_CLUSTERS, N_VXG), jnp.float32),
    ]

    def _half_specs(h):
        out_specs = [
            pl.BlockSpec((N_DONORS, N_CLUSTERS, _BLK), lambda j: (0, 0, j + h * hblk)),
            pl.BlockSpec((N_DONORS, N_CLUSTERS, _BLK), lambda j: (0, 0, j + h * hblk)),
        ]
        in_specs = [
            pl.BlockSpec((1, 1, _BLK), lambda j: (j + h * hblk, 0, 0)),
            pl.BlockSpec((N_CLUSTERS, _BLK), lambda j: (0, j + h * hblk)),
            pl.BlockSpec((_BLK, 4 * N_CLUSTERS), lambda j: (j, 0)),
            pl.BlockSpec((_BLK, 128), lambda j: (j, 0)),
            pl.BlockSpec((N_DONORS * N_CLUSTERS, N_GENES), lambda j: (0, 0)),
            pl.BlockSpec((N_DONORS, N_CLUSTERS), lambda j: (0, 0)),
        ]
        return in_specs, out_specs

    in_specs1, out_specs1 = _half_specs(0)
    expressed, elbo = pl.pallas_call(
        _main_body,
        grid=(hblk,),
        in_specs=in_specs1,
        out_specs=out_specs1,
        out_shape=out_shape,
    )(lidx, fc_log, mg_1, g_1, obs_bf, lib)

    in_specs2, out_specs2 = _half_specs(1)
    in_specs2 += [pl.BlockSpec(memory_space=pl.ANY),
                  pl.BlockSpec(memory_space=pl.ANY)]
    expressed, elbo = pl.pallas_call(
        _main_body,
        grid=(hblk,),
        in_specs=in_specs2,
        out_specs=out_specs2,
        out_shape=out_shape,
        input_output_aliases={6: 0, 7: 1},
    )(lidx, fc_log, mg_2, g_2, obs_bf, lib, expressed, elbo)
    return expressed, elbo
